# Initial kernel scaffold; baseline (speedup 1.0000x reference)
import jax
import jax.numpy as jnp
from jax.experimental import pallas as pl


def kernel(x, embed):
    raise NotImplementedError("write your pallas kernel here")


# speedup vs baseline: 1.5837x; 1.5837x over previous
import functools

import jax
import jax.numpy as jnp
from jax import lax
from jax.experimental import pallas as pl
from jax.experimental.pallas import tpu as pltpu
from jax.experimental.pallas import tpu_sc as plsc

BATCH = 4
SEQ = 4096
DIM = 2048
NTOK = BATCH * SEQ
NC = 2
NS = 16
NW = NC * NS
PER_W = NTOK // NW
WPR = SEQ // PER_W
LANES = 16
CH = 16
NCH = PER_W // CH
NVREG = PER_W // LANES


def _body(x_hbm, embed_hbm, out_hbm, x_v, idx_v, rows_v, sem):
    wid = lax.axis_index("s") * NC + lax.axis_index("c")
    row = wid // WPR
    ch = wid % WPR

    pltpu.sync_copy(x_hbm.at[pl.ds(row * SEQ, SEQ)], x_v)

    def pre_body(i, carry):
        v = x_v[pl.ds(i * LANES, LANES)]
        ones = jnp.where(v != 0, 1, 0).astype(jnp.int32)
        return carry + jnp.sum(ones)

    carry0 = lax.fori_loop(0, ch * NVREG, pre_body, jnp.int32(0))

    base = ch * PER_W

    def pos_body(j, carry):
        v = x_v[pl.ds(base + j * LANES, LANES)]
        ones = jnp.where(v != 0, 1, 0).astype(jnp.int32)
        cs = jnp.cumsum(ones) + carry
        idx_v[pl.ds(j * LANES, LANES)] = cs * ones
        return carry + jnp.sum(ones)

    lax.fori_loop(0, NVREG, pos_body, carry0)

    out_base = wid * PER_W

    def g_body(g, _):
        pltpu.async_copy(
            embed_hbm.at[idx_v.at[pl.ds(g * CH, CH)]], rows_v, sem
        ).wait()
        pltpu.sync_copy(rows_v, out_hbm.at[pl.ds(out_base + g * CH, CH)])
        return 0

    lax.fori_loop(0, NCH, g_body, 0)


@jax.jit
def kernel(x, embed):
    x_flat = x.reshape(NTOK)
    mesh = plsc.VectorSubcoreMesh(
        core_axis_name="c", subcore_axis_name="s", num_cores=NC,
        num_subcores=NS,
    )
    out = pl.kernel(
        _body,
        out_type=jax.ShapeDtypeStruct((NTOK, DIM), jnp.float32),
        mesh=mesh,
        compiler_params=pltpu.CompilerParams(needs_layout_passes=False),
        scratch_types=[
            pltpu.VMEM((SEQ,), jnp.int32),
            pltpu.VMEM((PER_W,), jnp.int32),
            pltpu.VMEM((CH, DIM), jnp.float32),
            pltpu.SemaphoreType.DMA,
        ],
    )(x_flat, embed)
    return out.reshape(BATCH, SEQ, DIM)


# speedup vs baseline: 1.8941x; 1.1960x over previous
import functools

import jax
import jax.numpy as jnp
from jax import lax
from jax.experimental import pallas as pl
from jax.experimental.pallas import tpu as pltpu
from jax.experimental.pallas import tpu_sc as plsc

BATCH = 4
SEQ = 4096
DIM = 2048
NTOK = BATCH * SEQ
NC = 2
NS = 16
NW = NC * NS
PER_W = NTOK // NW
WPR = SEQ // PER_W
LANES = 16
CH = 16
NCH = PER_W // CH
NVREG = PER_W // LANES


def _body(x_hbm, embed_hbm, out_hbm, x_v, idx_v, rows0_v, rows1_v, sem0,
          sem1):
    wid = lax.axis_index("s") * NC + lax.axis_index("c")
    row = wid // WPR
    ch = wid % WPR

    pltpu.sync_copy(x_hbm.at[pl.ds(row * SEQ, SEQ)], x_v)

    def pre_body(i, carry):
        v = x_v[pl.ds(i * LANES, LANES)]
        ones = jnp.where(v != 0, 1, 0).astype(jnp.int32)
        return carry + jnp.sum(ones)

    carry0 = lax.fori_loop(0, ch * NVREG, pre_body, jnp.int32(0))

    base = ch * PER_W

    def pos_body(j, carry):
        v = x_v[pl.ds(base + j * LANES, LANES)]
        ones = jnp.where(v != 0, 1, 0).astype(jnp.int32)
        cs = jnp.cumsum(ones) + carry
        idx_v[pl.ds(j * LANES, LANES)] = cs * ones
        return carry + jnp.sum(ones)

    lax.fori_loop(0, NVREG, pos_body, carry0)

    out_base = wid * PER_W

    def start_gather(g, buf, sem):
        pltpu.async_copy(embed_hbm.at[idx_v.at[pl.ds(g * CH, CH)]], buf, sem)

    def wait_gather(buf, sem):
        pltpu.make_async_copy(embed_hbm.at[idx_v.at[pl.ds(0, CH)]], buf,
                              sem).wait()

    def write_out(g, buf):
        pltpu.sync_copy(buf, out_hbm.at[pl.ds(out_base + g * CH, CH)])

    start_gather(0, rows0_v, sem0)
    start_gather(1, rows1_v, sem1)

    def g_body(k, _):
        g = 2 * k
        wait_gather(rows0_v, sem0)
        write_out(g, rows0_v)
        start_gather(g + 2, rows0_v, sem0)
        wait_gather(rows1_v, sem1)
        write_out(g + 1, rows1_v)
        start_gather(g + 3, rows1_v, sem1)
        return 0

    lax.fori_loop(0, NCH // 2 - 1, g_body, 0)
    wait_gather(rows0_v, sem0)
    write_out(NCH - 2, rows0_v)
    wait_gather(rows1_v, sem1)
    write_out(NCH - 1, rows1_v)


@jax.jit
def kernel(x, embed):
    x_flat = x.reshape(NTOK)
    mesh = plsc.VectorSubcoreMesh(
        core_axis_name="c", subcore_axis_name="s", num_cores=NC,
        num_subcores=NS,
    )
    out = pl.kernel(
        _body,
        out_type=jax.ShapeDtypeStruct((NTOK, DIM), jnp.float32),
        mesh=mesh,
        compiler_params=pltpu.CompilerParams(needs_layout_passes=False),
        scratch_types=[
            pltpu.VMEM((SEQ,), jnp.int32),
            pltpu.VMEM((PER_W,), jnp.int32),
            pltpu.VMEM((CH, DIM), jnp.float32),
            pltpu.VMEM((CH, DIM), jnp.float32),
            pltpu.SemaphoreType.DMA,
            pltpu.SemaphoreType.DMA,
        ],
    )(x_flat, embed)
    return out.reshape(BATCH, SEQ, DIM)
